# baseline (speedup 1.0000x reference)
import functools

import jax
import jax.numpy as jnp
from jax.experimental import pallas as pl
from jax.experimental.pallas import tpu as pltpu

D_MODEL = 2048
N_EXP = 16
BT = 512
NSLOT = 12
NSTEPS = 32
_CHUNKS = ([(s, 1, s) for s in range(4)]
           + [(4 + 4 * k, 4, 4 + k) for k in range(6)]
           + [(28 + j, 1, 10 + j) for j in range(4)])
_ISSUE_AT = {0: 0, 1: 0, 2: 0, 3: 0, 4: 0, 5: 0,
             6: 4, 7: 8, 8: 12, 9: 16, 10: 20, 11: 20, 12: 20, 13: 20}


def _router_block(x_hbm, wt_ref, w_ref, i_ref, acc_ref, aux_ref,
                  buf_ref, sem, *, inv_t):
    step = pl.program_id(0)

    def copy_desc(first_step, n_steps, sem_id):
        n = n_steps * BT
        return pltpu.make_async_copy(
            x_hbm.at[pl.ds(first_step * BT, n), :],
            buf_ref.at[pl.ds((first_step % NSLOT) * BT, n), :],
            sem.at[sem_id],
        )

    for issue_step in (0, 4, 8, 12, 16, 20):
        @pl.when(step == issue_step)
        def _(issue_step=issue_step):
            for first, n, sid in _CHUNKS:
                if _ISSUE_AT[sid] == issue_step:
                    copy_desc(first, n, sid).start()

    for first, n, sid in _CHUNKS:
        @pl.when(step == first)
        def _(first=first, n=n, sid=sid):
            copy_desc(first, n, sid).wait()

    off = jax.lax.rem(step, NSLOT) * BT
    xb = buf_ref[pl.ds(off, BT), :]
    logits = jnp.dot(xb, wt_ref[...], preferred_element_type=jnp.float32)

    ids = jax.lax.broadcasted_iota(jnp.int32, logits.shape, 1)
    m1 = jnp.max(logits, axis=1, keepdims=True)
    i1 = jnp.min(jnp.where(logits == m1, ids, N_EXP), axis=1, keepdims=True)
    masked = jnp.where(ids == i1, -jnp.inf, logits)
    m2 = jnp.max(masked, axis=1, keepdims=True)
    i2 = jnp.min(jnp.where(masked == m2, ids, N_EXP), axis=1, keepdims=True)

    t = jnp.exp(m2 - m1)
    w1 = 1.0 / (1.0 + t)
    w2 = t / (1.0 + t)
    w_ref[...] = jnp.concatenate([w1, w2], axis=1)
    i_ref[...] = jnp.concatenate([i1, i2], axis=1).astype(jnp.int32)

    p = jnp.exp(logits - m1)
    probs = p / jnp.sum(p, axis=1, keepdims=True)
    part = jnp.sum(probs, axis=0, keepdims=True)

    @pl.when(step == 0)
    def _():
        acc_ref[...] = jnp.zeros_like(acc_ref)

    acc_ref[...] += part

    @pl.when(step == NSTEPS - 1)
    def _():
        mean = acc_ref[...] * inv_t
        aux_ref[...] = jnp.sum(mean * mean, keepdims=True) * float(N_EXP)


def kernel(x, W):
    b, tt, d = x.shape
    total = b * tt
    xf = x.reshape(total, d)
    wt = W.T

    body = functools.partial(_router_block, inv_t=1.0 / total)
    weights, indices, _, aux = pl.pallas_call(
        body,
        grid=(NSTEPS,),
        in_specs=[
            pl.BlockSpec(memory_space=pl.ANY),
            pl.BlockSpec((d, N_EXP), lambda i: (0, 0)),
        ],
        out_specs=[
            pl.BlockSpec((BT, 2), lambda i: (i, 0)),
            pl.BlockSpec((BT, 2), lambda i: (i, 0)),
            pl.BlockSpec((1, N_EXP), lambda i: (0, 0)),
            pl.BlockSpec((1, 1), lambda i: (0, 0)),
        ],
        out_shape=[
            jax.ShapeDtypeStruct((total, 2), jnp.float32),
            jax.ShapeDtypeStruct((total, 2), jnp.int32),
            jax.ShapeDtypeStruct((1, N_EXP), jnp.float32),
            jax.ShapeDtypeStruct((1, 1), jnp.float32),
        ],
        scratch_shapes=[
            pltpu.VMEM((NSLOT * BT, D_MODEL), jnp.float32),
            pltpu.SemaphoreType.DMA((14,)),
        ],
    )(xf, wt)

    return (weights.reshape(b, tt, 2), indices.reshape(b, tt, 2), aux[0, 0])
